# baseline (speedup 1.0000x reference)
import functools

import jax
import jax.numpy as jnp
from jax import lax
from jax.experimental import pallas as pl
from jax.experimental.pallas import tpu as pltpu
from jax.experimental.pallas import tpu_sc as plsc

B = 16384
D = 32
V = 1000000
KP1 = 21
NC, NS = 2, 16
NW = NC * NS
PER_W = B // NW
DP = D // 2
CHUNK = 128
NCHUNK = PER_W // CHUNK
TOTCH = NW * NCHUNK


def _sc_scores(cemb_ch, cidx_ch, outr):
    mesh = plsc.VectorSubcoreMesh(
        core_axis_name="c", subcore_axis_name="s",
        num_cores=NC, num_subcores=NS)

    @functools.partial(
        pl.kernel,
        out_type=jax.ShapeDtypeStruct((TOTCH, KP1, CHUNK), jnp.float32),
        mesh=mesh,
        compiler_params=pltpu.CompilerParams(
            use_tc_tiling_on_sc=False, needs_layout_passes=False),
        scratch_types=[
            pltpu.VMEM((2, KP1, CHUNK), jnp.int32),
            pltpu.VMEM((2, CHUNK, D), jnp.float32),
            pltpu.VMEM((2, KP1, CHUNK, DP), jnp.int32),
            pltpu.VMEM((2, KP1, CHUNK), jnp.float32),
            pltpu.SemaphoreType.DMA,
            pltpu.SemaphoreType.DMA,
        ],
    )
    def k(cemb_hbm, cidx_hbm, outr_hbm, out_hbm,
          cidx_v, crows_v, orows_v, sc_v, sem0, sem1):
        wid = lax.axis_index("s") * NC + lax.axis_index("c")
        iota = lax.iota(jnp.int32, 16)
        sems = (sem0, sem1)

        def stage_and_fire(c, b):
            ch = wid * NCHUNK + c
            pltpu.sync_copy(cidx_hbm.at[ch], cidx_v.at[b])
            pltpu.async_copy(cemb_hbm.at[ch], crows_v.at[b], sems[b])
            for j in range(KP1):
                pltpu.async_copy(
                    outr_hbm.at[cidx_v.at[b, j]], orows_v.at[b, j],
                    sems[b])

        def wait_gathers(c, b):
            ch = wid * NCHUNK + c
            pltpu.make_async_copy(
                cemb_hbm.at[ch], crows_v.at[b], sems[b]).wait()
            for j in range(KP1):
                pltpu.make_async_copy(
                    outr_hbm.at[cidx_v.at[b, j]], orows_v.at[b, j],
                    sems[b]).wait()

        def compute(c, b):
            def g_body(g, _):
                r16 = g * 16 + iota
                bb = jnp.full((16,), b, jnp.int32)
                ccols = [
                    plsc.load_gather(
                        crows_v, [bb, r16, jnp.full((16,), d, jnp.int32)])
                    for d in range(D)
                ]
                for j in range(KP1):
                    jj = jnp.full((16,), j, jnp.int32)
                    s = None
                    for d2 in range(DP):
                        u = plsc.load_gather(
                            orows_v,
                            [bb, jj, r16, jnp.full((16,), d2, jnp.int32)])
                        lo, hi = plsc.unpack(
                            plsc.bitcast(u, jnp.bfloat16),
                            format=plsc.PackFormat.INTERLEAVED)
                        t = (ccols[2 * d2] * lo.astype(jnp.float32)
                             + ccols[2 * d2 + 1] * hi.astype(jnp.float32))
                        s = t if s is None else s + t
                    if j:
                        s = -s
                    sc_v[b, j, pl.ds(g * 16, 16)] = s
                return 0

            lax.fori_loop(0, CHUNK // 16, g_body, 0)
            pltpu.sync_copy(sc_v.at[b], out_hbm.at[wid * NCHUNK + c])

        stage_and_fire(0, 0)

        def pair_body(c2, _):
            for b in range(2):
                c = c2 * 2 + b
                wait_gathers(c, b)

                @pl.when(c + 1 < NCHUNK)
                def _():
                    stage_and_fire(c + 1, 1 - b)

                compute(c, b)
            return 0

        lax.fori_loop(0, NCHUNK // 2, pair_body, 0)

    return k(cemb_ch, cidx_ch, outr)


def _tc_loss(scores):
    x2 = scores.reshape(B * KP1 // 128, 128)

    def body(x_ref, o_ref):
        x = x_ref[...]
        ls = jnp.minimum(x, 0.0) - jnp.log1p(jnp.exp(-jnp.abs(x)))
        o_ref[0, 0] = -jnp.sum(ls) * (1.0 / B)

    out = pl.pallas_call(
        body,
        out_shape=jax.ShapeDtypeStruct((1, 1), jnp.float32),
        out_specs=pl.BlockSpec(memory_space=pltpu.SMEM),
    )(x2)
    return out[0, 0]


def _pack_out(t):
    u = jax.lax.bitcast_convert_type(t, jnp.uint32)
    r = (u + 0x7FFF + ((u >> 16) & 1)) >> 16
    packed = r[:, 0::2] | (r[:, 1::2] << 16)
    return jax.lax.bitcast_convert_type(packed, jnp.int32)


def kernel(center, context, negatives, in_embed, out_embed):
    cidx = jnp.concatenate([context[:, None], negatives], axis=1)
    cidx_ch = cidx.reshape(TOTCH, CHUNK, KP1).transpose(0, 2, 1)
    cemb_ch = jnp.take(in_embed, center, axis=0,
                       mode="clip").reshape(TOTCH, CHUNK, D)
    scores = _sc_scores(cemb_ch, cidx_ch, _pack_out(out_embed))
    return _tc_loss(scores)


# speedup vs baseline: 11.4307x; 11.4307x over previous
import functools

import jax
import jax.numpy as jnp
from jax import lax
from jax.experimental import pallas as pl
from jax.experimental.pallas import tpu as pltpu
from jax.experimental.pallas import tpu_sc as plsc

B = 16384
D = 32
V = 1000000
KP1 = 21
NC, NS = 2, 16
NW = NC * NS
PER_W = B // NW
CHUNK = 64
NCHUNK = PER_W // CHUNK
TOTCH = NW * NCHUNK


def _sc_scores(cemb_ch, cidx_ch, outr):
    mesh = plsc.VectorSubcoreMesh(
        core_axis_name="c", subcore_axis_name="s",
        num_cores=NC, num_subcores=NS)

    @functools.partial(
        pl.kernel,
        out_type=jax.ShapeDtypeStruct((TOTCH, KP1, CHUNK), jnp.float32),
        mesh=mesh,
        compiler_params=pltpu.CompilerParams(
            use_tc_tiling_on_sc=False, needs_layout_passes=False),
        scratch_types=[
            pltpu.VMEM((2, KP1, CHUNK), jnp.int32),
            pltpu.VMEM((2, CHUNK, D), jnp.float32),
            pltpu.VMEM((2, KP1, CHUNK, D), jnp.float32),
            pltpu.VMEM((2, KP1, CHUNK), jnp.float32),
            pltpu.SemaphoreType.DMA,
            pltpu.SemaphoreType.DMA,
        ],
    )
    def k(cemb_hbm, cidx_hbm, outr_hbm, out_hbm,
          cidx_v, crows_v, orows_v, sc_v, sem0, sem1):
        wid = lax.axis_index("s") * NC + lax.axis_index("c")
        iota = lax.iota(jnp.int32, 16)
        sems = (sem0, sem1)

        def stage_and_fire(c, b):
            ch = wid * NCHUNK + c
            pltpu.sync_copy(cidx_hbm.at[ch], cidx_v.at[b])
            pltpu.async_copy(cemb_hbm.at[ch], crows_v.at[b], sems[b])
            for j in range(KP1):
                pltpu.async_copy(
                    outr_hbm.at[cidx_v.at[b, j]], orows_v.at[b, j],
                    sems[b])

        def wait_gathers(c, b):
            ch = wid * NCHUNK + c
            pltpu.make_async_copy(
                cemb_hbm.at[ch], crows_v.at[b], sems[b]).wait()
            for j in range(KP1):
                pltpu.make_async_copy(
                    outr_hbm.at[cidx_v.at[b, j]], orows_v.at[b, j],
                    sems[b]).wait()

        def compute(c, b):
            def g_body(g, _):
                r16 = g * 16 + iota
                bb = jnp.full((16,), b, jnp.int32)
                ccols = [
                    plsc.load_gather(
                        crows_v, [bb, r16, jnp.full((16,), d, jnp.int32)])
                    for d in range(D)
                ]
                for j in range(KP1):
                    jj = jnp.full((16,), j, jnp.int32)
                    s = ccols[0] * plsc.load_gather(
                        orows_v, [bb, jj, r16, jnp.full((16,), 0, jnp.int32)])
                    for d in range(1, D):
                        s = s + ccols[d] * plsc.load_gather(
                            orows_v,
                            [bb, jj, r16, jnp.full((16,), d, jnp.int32)])
                    if j:
                        s = -s
                    sc_v[b, j, pl.ds(g * 16, 16)] = s
                return 0

            lax.fori_loop(0, CHUNK // 16, g_body, 0)
            pltpu.sync_copy(sc_v.at[b], out_hbm.at[wid * NCHUNK + c])

        stage_and_fire(0, 0)

        def pair_body(c2, _):
            for b in range(2):
                c = c2 * 2 + b
                wait_gathers(c, b)

                @pl.when(c + 1 < NCHUNK)
                def _():
                    stage_and_fire(c + 1, 1 - b)

                compute(c, b)
            return 0

        lax.fori_loop(0, NCHUNK // 2, pair_body, 0)

    return k(cemb_ch, cidx_ch, outr)


def _tc_loss(scores):
    x2 = scores.reshape(B * KP1 // 128, 128)

    def body(x_ref, o_ref):
        x = x_ref[...]
        ls = jnp.minimum(x, 0.0) - jnp.log1p(jnp.exp(-jnp.abs(x)))
        o_ref[0, 0] = -jnp.sum(ls) * (1.0 / B)

    out = pl.pallas_call(
        body,
        out_shape=jax.ShapeDtypeStruct((1, 1), jnp.float32),
        out_specs=pl.BlockSpec(memory_space=pltpu.SMEM),
    )(x2)
    return out[0, 0]


def kernel(center, context, negatives, in_embed, out_embed):
    cidx = jnp.concatenate([context[:, None], negatives], axis=1)
    cidx_ch = cidx.reshape(TOTCH, CHUNK, KP1).transpose(0, 2, 1)
    cemb_ch = jnp.take(in_embed, center, axis=0,
                       mode="clip").reshape(TOTCH, CHUNK, D)
    scores = _sc_scores(cemb_ch, cidx_ch, out_embed)
    return _tc_loss(scores)
